# baseline (speedup 1.0000x reference)
import functools
import numpy as np
import jax
import jax.numpy as jnp
from jax import lax
from jax.experimental import pallas as pl
from jax.experimental.pallas import tpu as pltpu
from jax.experimental.pallas import tpu_sc as plsc

_N = 1000000
_M = 100
_K = 11
_L = 16
_NCH = 7
_NW = 32
_CHUNK = 31248
_TAIL = _N - _NW * _CHUNK
_TAIL_BASE = _NW * _CHUNK
_UNROLL = 9
_NITER = _CHUNK // (_L * _UNROLL)
_NITER1 = 108
_SPLIT = _NITER1 * _L * _UNROLL
_NITER2 = _NITER - _NITER1
_REST = _CHUNK - _SPLIT
_INF = float(np.inf)

_PERM100 = np.asarray([
    168450, 920172, 395105, 263872, 221109, 78064, 858077, 520158, 330145,
    555220, 766525, 15925, 447126, 550083, 583420, 413555, 662564, 151649,
    528156, 786375, 391712, 285245, 201641, 174004, 8595, 602412, 882598,
    554843, 78597, 752653, 318832, 130130, 118778, 291031, 851275, 141787,
    69026, 509543, 678130, 987805, 152648, 243323, 177380, 689120, 816119,
    177489, 745975, 143737, 943136, 266152, 786620, 853560, 969641, 861815,
    244708, 978776, 248512, 45655, 23208, 370197, 107389, 588445, 468632,
    950894, 196375, 417740, 909312, 483740, 709397, 199205, 358812, 550817,
    671241, 541311, 454740, 76131, 105319, 123046, 186913, 746742, 891006,
    952967, 678045, 6549, 906382, 491437, 728325, 614679, 750116, 162766,
    880843, 815723, 800078, 455911, 716915, 877054, 313050, 164116, 158191,
    157157,
], dtype=np.int32)

_IDX_PAD = np.zeros((128,), dtype=np.int32)
_IDX_PAD[:_M] = _PERM100


def _vsqrt(x):
    bits = lax.bitcast_convert_type(x, jnp.int32)
    s = lax.bitcast_convert_type(
        (bits >> 1) + jnp.int32(0x1FBD1DF5), jnp.float32)
    for _ in range(3):
        s = jnp.float32(0.5) * (s + x / s)
    return s


def _lex_take(k, i, k2, i2):
    take = (k2 < k) | ((k2 == k) & (i2 < i))
    return jnp.where(take, k2, k), jnp.where(take, i2, i)


def _sc_body(yp_hbm, yt_hbm, cxy_hbm, idx_hbm, out_hbm,
             yp_buf, yt_buf, tyb, ttb, idx_v, cxy_v,
             yps, yts, rsub, fscr, iscr, orow, sem0, sem1, sem2):
    c = lax.axis_index("c")
    s = lax.axis_index("s")
    w = s * 2 + c
    base = w * _CHUNK

    pltpu.sync_copy(idx_hbm, idx_v)
    g0 = pltpu.async_copy(yp_hbm.at[idx_v], yps, sem2)
    g1 = pltpu.async_copy(yt_hbm.at[idx_v], yts, sem2)
    pltpu.sync_copy(cxy_hbm, cxy_v)

    cp0 = pltpu.async_copy(yp_hbm.at[pl.ds(base, _SPLIT)],
                           yp_buf.at[pl.ds(0, _SPLIT)], sem0)
    cp1 = pltpu.async_copy(yt_hbm.at[pl.ds(base, _SPLIT)],
                           yt_buf.at[pl.ds(0, _SPLIT)], sem0)
    cp2 = pltpu.async_copy(yp_hbm.at[pl.ds(base + _SPLIT, _REST)],
                           yp_buf.at[pl.ds(_SPLIT, _REST)], sem1)
    cp3 = pltpu.async_copy(yt_hbm.at[pl.ds(base + _SPLIT, _REST)],
                           yt_buf.at[pl.ds(_SPLIT, _REST)], sem1)

    zero16 = jnp.zeros((_L,), jnp.float32)
    for k in range(_TAIL // _L):
        tyb[pl.ds(k * _L, _L)] = zero16
        ttb[pl.ds(k * _L, _L)] = zero16

    @pl.when(w == _NW - 1)
    def _():
        pltpu.sync_copy(yp_hbm.at[pl.ds(_TAIL_BASE, _TAIL)], tyb)
        pltpu.sync_copy(yt_hbm.at[pl.ds(_TAIL_BASE, _TAIL)], ttb)

    g0.wait()
    g1.wait()

    iota = lax.iota(jnp.int32, _L)
    zeros_i = jnp.zeros((_L,), jnp.int32)
    ones_i = jnp.ones((_L,), jnp.int32)
    cxv, cyv, gidx = [], [], []
    for ci in range(_NCH):
        sl = pl.ds(ci * _L, _L)
        rsub[sl] = yps[sl] - yts[sl]
        gi = iota + ci * _L
        gidx.append(gi)
        cxv.append(plsc.load_gather(cxy_v, [gi, zeros_i]))
        cyv.append(plsc.load_gather(cxy_v, [gi, ones_i]))
    rsub[pl.ds(_NCH * _L, _L)] = zero16

    def bf_sum(v):
        for sh in (8, 4, 2, 1):
            fscr[...] = v
            v = v + plsc.load_gather(fscr, [iota ^ sh])
        return v

    def bf_lex_min(k, i):
        for sh in (8, 4, 2, 1):
            fscr[...] = k
            iscr[...] = i
            idx = iota ^ sh
            k2 = plsc.load_gather(fscr, [idx])
            i2 = plsc.load_gather(iscr, [idx])
            k, i = _lex_take(k, i, k2, i2)
        return k, i

    def splat(ref, j):
        return plsc.load_gather(ref, [jnp.broadcast_to(j, (_L,))])

    def knn_point(t, pen_acc):
        ii = w * 4 + t
        ii_v = jnp.broadcast_to(ii, (_L,))
        cxi = plsc.load_gather(cxy_v, [ii_v, zeros_i])
        cyi = plsc.load_gather(cxy_v, [ii_v, ones_i])
        rii = splat(rsub, ii)

        dch = []
        for ci in range(_NCH):
            dx = cxv[ci] - cxi
            dy = cyv[ci] - cyi
            d2 = dx * dx + dy * dy
            dch.append(jnp.where(gidx[ci] < _M, d2, _INF))

        def round_body(rnd, st):
            dchs, dvec, rvec = st
            mk, mi = dchs[0], gidx[0]
            for ci in range(1, _NCH):
                mk, mi = _lex_take(mk, mi, dchs[ci], gidx[ci])
            mk, mi = bf_lex_min(mk, mi)
            rstar = splat(rsub, mi)
            lane_hit = iota == rnd
            dvec = jnp.where(lane_hit, mk, dvec)
            rvec = jnp.where(lane_hit, rstar, rvec)
            new_dchs = tuple(
                jnp.where(gidx[ci] == mi, _INF, dchs[ci])
                for ci in range(_NCH))
            return (new_dchs, dvec, rvec)

        zeros = jnp.zeros((_L,), jnp.float32)
        _, dvec, rvec = lax.fori_loop(
            0, _K, round_body, (tuple(dch), zeros, zeros))

        sd = _vsqrt(dvec)
        fscr[...] = sd
        dmax = plsc.load_gather(fscr, [jnp.broadcast_to(jnp.int32(_K - 1),
                                                        (_L,))])
        norm = sd / (dmax + jnp.float32(1e-8))
        penv = jnp.abs(jnp.abs(rvec - rii) - norm)
        lane_ok = ((iota >= 1) & (iota < _K)).astype(jnp.float32)
        return pen_acc + penv * lane_ok

    pen_vec = lax.fori_loop(0, 4, knn_point, jnp.zeros((_L,), jnp.float32))
    pen_vec = bf_sum(pen_vec)
    pen_gate = jnp.where(w < (_M + 3) // 4, jnp.float32(1.0), jnp.float32(0.0))

    def mse_body(j, accs):
        b = j * (_L * _UNROLL)
        out = []
        for u in range(_UNROLL):
            d = (yp_buf[pl.ds(b + u * _L, _L)]
                 - yt_buf[pl.ds(b + u * _L, _L)])
            out.append(accs[u] + d * d)
        return tuple(out)

    zeros = jnp.zeros((_L,), jnp.float32)
    cp0.wait()
    cp1.wait()
    accs = lax.fori_loop(0, _NITER1, mse_body, (zeros,) * _UNROLL)
    cp2.wait()
    cp3.wait()
    accs = lax.fori_loop(_NITER1, _NITER, mse_body, accs)
    acc = accs[0]
    for u in range(1, _UNROLL):
        acc = acc + accs[u]
    for k in range(_TAIL // _L):
        dt = tyb[pl.ds(k * _L, _L)] - ttb[pl.ds(k * _L, _L)]
        acc = acc + dt * dt

    lane0 = (iota == 0).astype(jnp.float32)
    contrib = acc * jnp.float32(0.8 / _N)
    contrib = contrib + pen_vec * lane0 * (
        pen_gate * jnp.float32(0.2 / (_M * (_K - 1))))
    orow[...] = contrib
    pltpu.sync_copy(orow, out_hbm.at[w])


_sc_kernel = functools.partial(
    pl.kernel,
    mesh=plsc.VectorSubcoreMesh(core_axis_name="c", subcore_axis_name="s"),
    compiler_params=pltpu.CompilerParams(needs_layout_passes=False),
    out_type=jax.ShapeDtypeStruct((_NW, _L), jnp.float32),
    scratch_types=[
        pltpu.VMEM((_CHUNK,), jnp.float32),
        pltpu.VMEM((_CHUNK,), jnp.float32),
        pltpu.VMEM((_TAIL,), jnp.float32),
        pltpu.VMEM((_TAIL,), jnp.float32),
        pltpu.VMEM((128,), jnp.int32),
        pltpu.VMEM((128, 2), jnp.float32),
        pltpu.VMEM((128,), jnp.float32),
        pltpu.VMEM((128,), jnp.float32),
        pltpu.VMEM((128,), jnp.float32),
        pltpu.VMEM((_L,), jnp.float32),
        pltpu.VMEM((_L,), jnp.int32),
        pltpu.VMEM((_L,), jnp.float32),
        pltpu.SemaphoreType.DMA,
        pltpu.SemaphoreType.DMA,
        pltpu.SemaphoreType.DMA,
    ],
)(_sc_body)


def kernel(y_pred, y_true, coordinates):
    idxc = jnp.asarray(_IDX_PAD)
    cxy = coordinates[idxc]
    partials = _sc_kernel(y_pred, y_true, cxy, idxc)
    return jnp.sum(partials)


# speedup vs baseline: 1.0004x; 1.0004x over previous
import functools
import numpy as np
import jax
import jax.numpy as jnp
from jax import lax
from jax.experimental import pallas as pl
from jax.experimental.pallas import tpu as pltpu
from jax.experimental.pallas import tpu_sc as plsc

_N = 1000000
_M = 100
_K = 11
_L = 16
_NCH = 7
_NW = 32
_CHUNK = 31248
_TAIL = _N - _NW * _CHUNK
_TAIL_BASE = _NW * _CHUNK
_UNROLL = 9
_NITER = _CHUNK // (_L * _UNROLL)
_NITER1 = 108
_SPLIT = _NITER1 * _L * _UNROLL
_NITER2 = _NITER - _NITER1
_REST = _CHUNK - _SPLIT
_INF = float(np.inf)

_PERM100 = np.asarray([
    168450, 920172, 395105, 263872, 221109, 78064, 858077, 520158, 330145,
    555220, 766525, 15925, 447126, 550083, 583420, 413555, 662564, 151649,
    528156, 786375, 391712, 285245, 201641, 174004, 8595, 602412, 882598,
    554843, 78597, 752653, 318832, 130130, 118778, 291031, 851275, 141787,
    69026, 509543, 678130, 987805, 152648, 243323, 177380, 689120, 816119,
    177489, 745975, 143737, 943136, 266152, 786620, 853560, 969641, 861815,
    244708, 978776, 248512, 45655, 23208, 370197, 107389, 588445, 468632,
    950894, 196375, 417740, 909312, 483740, 709397, 199205, 358812, 550817,
    671241, 541311, 454740, 76131, 105319, 123046, 186913, 746742, 891006,
    952967, 678045, 6549, 906382, 491437, 728325, 614679, 750116, 162766,
    880843, 815723, 800078, 455911, 716915, 877054, 313050, 164116, 158191,
    157157,
], dtype=np.int32)

_IDX_PAD = np.zeros((128,), dtype=np.int32)
_IDX_PAD[:_M] = _PERM100


def _vsqrt(x):
    bits = lax.bitcast_convert_type(x, jnp.int32)
    s = lax.bitcast_convert_type(
        (bits >> 1) + jnp.int32(0x1FBD1DF5), jnp.float32)
    for _ in range(3):
        s = jnp.float32(0.5) * (s + x / s)
    return s


def _lex_take(k, i, k2, i2):
    take = (k2 < k) | ((k2 == k) & (i2 < i))
    return jnp.where(take, k2, k), jnp.where(take, i2, i)


def _sc_body(yp_hbm, yt_hbm, cxy_hbm, idx_hbm, out_hbm,
             yp_buf, yt_buf, tyb, ttb, idx_v, cxy_v,
             yps, yts, rsub, fscr, iscr, orow, sem0, sem1, sem2):
    c = lax.axis_index("c")
    s = lax.axis_index("s")
    w = s * 2 + c
    base = w * _CHUNK

    pltpu.sync_copy(idx_hbm, idx_v)
    g0 = pltpu.async_copy(yp_hbm.at[idx_v], yps, sem2)
    g1 = pltpu.async_copy(yt_hbm.at[idx_v], yts, sem2)
    pltpu.sync_copy(cxy_hbm, cxy_v)

    cp0 = pltpu.async_copy(yp_hbm.at[pl.ds(base, _CHUNK)], yp_buf, sem0)
    cp1 = pltpu.async_copy(yt_hbm.at[pl.ds(base, _CHUNK)], yt_buf, sem1)

    zero16 = jnp.zeros((_L,), jnp.float32)
    for k in range(_TAIL // _L):
        tyb[pl.ds(k * _L, _L)] = zero16
        ttb[pl.ds(k * _L, _L)] = zero16

    @pl.when(w == _NW - 1)
    def _():
        pltpu.sync_copy(yp_hbm.at[pl.ds(_TAIL_BASE, _TAIL)], tyb)
        pltpu.sync_copy(yt_hbm.at[pl.ds(_TAIL_BASE, _TAIL)], ttb)

    g0.wait()
    g1.wait()

    iota = lax.iota(jnp.int32, _L)
    zeros_i = jnp.zeros((_L,), jnp.int32)
    ones_i = jnp.ones((_L,), jnp.int32)
    cxv, cyv, gidx = [], [], []
    for ci in range(_NCH):
        sl = pl.ds(ci * _L, _L)
        rsub[sl] = yps[sl] - yts[sl]
        gi = iota + ci * _L
        gidx.append(gi)
        cxv.append(plsc.load_gather(cxy_v, [gi, zeros_i]))
        cyv.append(plsc.load_gather(cxy_v, [gi, ones_i]))
    rsub[pl.ds(_NCH * _L, _L)] = zero16

    def bf_sum(v):
        for sh in (8, 4, 2, 1):
            fscr[...] = v
            v = v + plsc.load_gather(fscr, [iota ^ sh])
        return v

    def bf_lex_min(k, i):
        for sh in (8, 4, 2, 1):
            fscr[...] = k
            iscr[...] = i
            idx = iota ^ sh
            k2 = plsc.load_gather(fscr, [idx])
            i2 = plsc.load_gather(iscr, [idx])
            k, i = _lex_take(k, i, k2, i2)
        return k, i

    def splat(ref, j):
        return plsc.load_gather(ref, [jnp.broadcast_to(j, (_L,))])

    def knn_point(t, pen_acc):
        ii = w * 4 + t
        ii_v = jnp.broadcast_to(ii, (_L,))
        cxi = plsc.load_gather(cxy_v, [ii_v, zeros_i])
        cyi = plsc.load_gather(cxy_v, [ii_v, ones_i])
        rii = splat(rsub, ii)

        dch = []
        for ci in range(_NCH):
            dx = cxv[ci] - cxi
            dy = cyv[ci] - cyi
            d2 = dx * dx + dy * dy
            dch.append(jnp.where(gidx[ci] < _M, d2, _INF))

        def round_body(rnd, st):
            dchs, dvec, rvec = st
            mk, mi = dchs[0], gidx[0]
            for ci in range(1, _NCH):
                mk, mi = _lex_take(mk, mi, dchs[ci], gidx[ci])
            mk, mi = bf_lex_min(mk, mi)
            rstar = splat(rsub, mi)
            lane_hit = iota == rnd
            dvec = jnp.where(lane_hit, mk, dvec)
            rvec = jnp.where(lane_hit, rstar, rvec)
            new_dchs = tuple(
                jnp.where(gidx[ci] == mi, _INF, dchs[ci])
                for ci in range(_NCH))
            return (new_dchs, dvec, rvec)

        zeros = jnp.zeros((_L,), jnp.float32)
        _, dvec, rvec = lax.fori_loop(
            0, _K, round_body, (tuple(dch), zeros, zeros))

        sd = _vsqrt(dvec)
        fscr[...] = sd
        dmax = plsc.load_gather(fscr, [jnp.broadcast_to(jnp.int32(_K - 1),
                                                        (_L,))])
        norm = sd / (dmax + jnp.float32(1e-8))
        penv = jnp.abs(jnp.abs(rvec - rii) - norm)
        lane_ok = ((iota >= 1) & (iota < _K)).astype(jnp.float32)
        return pen_acc + penv * lane_ok

    pen_vec = lax.fori_loop(0, 4, knn_point, jnp.zeros((_L,), jnp.float32))
    pen_vec = bf_sum(pen_vec)
    pen_gate = jnp.where(w < (_M + 3) // 4, jnp.float32(1.0), jnp.float32(0.0))

    def mse_body(j, accs):
        b = j * (_L * _UNROLL)
        out = []
        for u in range(_UNROLL):
            d = (yp_buf[pl.ds(b + u * _L, _L)]
                 - yt_buf[pl.ds(b + u * _L, _L)])
            out.append(accs[u] + d * d)
        return tuple(out)

    zeros = jnp.zeros((_L,), jnp.float32)
    cp0.wait()
    cp1.wait()
    accs = lax.fori_loop(0, _NITER, mse_body, (zeros,) * _UNROLL)
    acc = accs[0]
    for u in range(1, _UNROLL):
        acc = acc + accs[u]
    for k in range(_TAIL // _L):
        dt = tyb[pl.ds(k * _L, _L)] - ttb[pl.ds(k * _L, _L)]
        acc = acc + dt * dt

    lane0 = (iota == 0).astype(jnp.float32)
    contrib = acc * jnp.float32(0.8 / _N)
    contrib = contrib + pen_vec * lane0 * (
        pen_gate * jnp.float32(0.2 / (_M * (_K - 1))))
    orow[...] = contrib
    pltpu.sync_copy(orow, out_hbm.at[w])


_sc_kernel = functools.partial(
    pl.kernel,
    mesh=plsc.VectorSubcoreMesh(core_axis_name="c", subcore_axis_name="s"),
    compiler_params=pltpu.CompilerParams(needs_layout_passes=False),
    out_type=jax.ShapeDtypeStruct((_NW, _L), jnp.float32),
    scratch_types=[
        pltpu.VMEM((_CHUNK,), jnp.float32),
        pltpu.VMEM((_CHUNK,), jnp.float32),
        pltpu.VMEM((_TAIL,), jnp.float32),
        pltpu.VMEM((_TAIL,), jnp.float32),
        pltpu.VMEM((128,), jnp.int32),
        pltpu.VMEM((128, 2), jnp.float32),
        pltpu.VMEM((128,), jnp.float32),
        pltpu.VMEM((128,), jnp.float32),
        pltpu.VMEM((128,), jnp.float32),
        pltpu.VMEM((_L,), jnp.float32),
        pltpu.VMEM((_L,), jnp.int32),
        pltpu.VMEM((_L,), jnp.float32),
        pltpu.SemaphoreType.DMA,
        pltpu.SemaphoreType.DMA,
        pltpu.SemaphoreType.DMA,
    ],
)(_sc_body)


def kernel(y_pred, y_true, coordinates):
    idxc = jnp.asarray(_IDX_PAD)
    cxy = coordinates[idxc]
    partials = _sc_kernel(y_pred, y_true, cxy, idxc)
    return jnp.sum(partials)


# speedup vs baseline: 1.0610x; 1.0606x over previous
import functools
import numpy as np
import jax
import jax.numpy as jnp
from jax import lax
from jax.experimental import pallas as pl
from jax.experimental.pallas import tpu as pltpu
from jax.experimental.pallas import tpu_sc as plsc

_N = 1000000
_M = 100
_K = 11
_L = 16
_NCH = 7
_NW = 32
_CHUNK = 31248
_TAIL = _N - _NW * _CHUNK
_TAIL_BASE = _NW * _CHUNK
_UNROLL = 9
_NITER = _CHUNK // (_L * _UNROLL)
_NITER1 = 108
_SPLIT = _NITER1 * _L * _UNROLL
_NITER2 = _NITER - _NITER1
_REST = _CHUNK - _SPLIT
_INF = float(np.inf)

_PERM100 = np.asarray([
    168450, 920172, 395105, 263872, 221109, 78064, 858077, 520158, 330145,
    555220, 766525, 15925, 447126, 550083, 583420, 413555, 662564, 151649,
    528156, 786375, 391712, 285245, 201641, 174004, 8595, 602412, 882598,
    554843, 78597, 752653, 318832, 130130, 118778, 291031, 851275, 141787,
    69026, 509543, 678130, 987805, 152648, 243323, 177380, 689120, 816119,
    177489, 745975, 143737, 943136, 266152, 786620, 853560, 969641, 861815,
    244708, 978776, 248512, 45655, 23208, 370197, 107389, 588445, 468632,
    950894, 196375, 417740, 909312, 483740, 709397, 199205, 358812, 550817,
    671241, 541311, 454740, 76131, 105319, 123046, 186913, 746742, 891006,
    952967, 678045, 6549, 906382, 491437, 728325, 614679, 750116, 162766,
    880843, 815723, 800078, 455911, 716915, 877054, 313050, 164116, 158191,
    157157,
], dtype=np.int32)

_IDX_PAD = np.zeros((128,), dtype=np.int32)
_IDX_PAD[:_M] = _PERM100


def _vsqrt(x):
    bits = lax.bitcast_convert_type(x, jnp.int32)
    s = lax.bitcast_convert_type(
        (bits >> 1) + jnp.int32(0x1FBD1DF5), jnp.float32)
    for _ in range(3):
        s = jnp.float32(0.5) * (s + x / s)
    return s


def _lex_take(k, i, k2, i2):
    take = (k2 < k) | ((k2 == k) & (i2 < i))
    return jnp.where(take, k2, k), jnp.where(take, i2, i)


def _sc_body(yp_hbm, yt_hbm, cxy_hbm, idx_hbm, out_hbm,
             yp_buf, yt_buf, tyb, ttb, idx_v,
             yps, yts, cxs, cys, rsub, fscr, iscr, orow, sem0, sem1, sem2):
    c = lax.axis_index("c")
    s = lax.axis_index("s")
    w = s * 2 + c
    base = w * _CHUNK

    pltpu.sync_copy(idx_hbm, idx_v)
    g0 = pltpu.async_copy(yp_hbm.at[idx_v], yps, sem2)
    g1 = pltpu.async_copy(yt_hbm.at[idx_v], yts, sem2)
    pltpu.sync_copy(cxy_hbm.at[0], cxs)
    pltpu.sync_copy(cxy_hbm.at[1], cys)

    cp0 = pltpu.async_copy(yp_hbm.at[pl.ds(base, _CHUNK)], yp_buf, sem0)
    cp1 = pltpu.async_copy(yt_hbm.at[pl.ds(base, _CHUNK)], yt_buf, sem1)

    zero16 = jnp.zeros((_L,), jnp.float32)
    for k in range(_TAIL // _L):
        tyb[pl.ds(k * _L, _L)] = zero16
        ttb[pl.ds(k * _L, _L)] = zero16

    @pl.when(w == _NW - 1)
    def _():
        pltpu.sync_copy(yp_hbm.at[pl.ds(_TAIL_BASE, _TAIL)], tyb)
        pltpu.sync_copy(yt_hbm.at[pl.ds(_TAIL_BASE, _TAIL)], ttb)

    g0.wait()
    g1.wait()

    iota = lax.iota(jnp.int32, _L)
    cxv, cyv, gidx = [], [], []
    for ci in range(_NCH):
        sl = pl.ds(ci * _L, _L)
        rsub[sl] = yps[sl] - yts[sl]
        gidx.append(iota + ci * _L)
        cxv.append(cxs[sl])
        cyv.append(cys[sl])
    rsub[pl.ds(_NCH * _L, _L)] = zero16

    def bf_sum(v):
        for sh in (8, 4, 2, 1):
            fscr[...] = v
            v = v + plsc.load_gather(fscr, [iota ^ sh])
        return v

    def bf_lex_min(k, i):
        for sh in (8, 4, 2, 1):
            fscr[...] = k
            iscr[...] = i
            idx = iota ^ sh
            k2 = plsc.load_gather(fscr, [idx])
            i2 = plsc.load_gather(iscr, [idx])
            k, i = _lex_take(k, i, k2, i2)
        return k, i

    def splat(ref, j):
        return plsc.load_gather(ref, [jnp.broadcast_to(j, (_L,))])

    def knn_point(t, pen_acc):
        ii = w * 4 + t
        cxi = splat(cxs, ii)
        cyi = splat(cys, ii)
        rii = splat(rsub, ii)

        dch = []
        for ci in range(_NCH):
            dx = cxv[ci] - cxi
            dy = cyv[ci] - cyi
            d2 = dx * dx + dy * dy
            dch.append(jnp.where(gidx[ci] < _M, d2, _INF))

        def round_body(rnd, st):
            dchs, dvec, rvec = st
            mk, mi = dchs[0], gidx[0]
            for ci in range(1, _NCH):
                mk, mi = _lex_take(mk, mi, dchs[ci], gidx[ci])
            mk, mi = bf_lex_min(mk, mi)
            rstar = splat(rsub, mi)
            lane_hit = iota == rnd
            dvec = jnp.where(lane_hit, mk, dvec)
            rvec = jnp.where(lane_hit, rstar, rvec)
            new_dchs = tuple(
                jnp.where(gidx[ci] == mi, _INF, dchs[ci])
                for ci in range(_NCH))
            return (new_dchs, dvec, rvec)

        zeros = jnp.zeros((_L,), jnp.float32)
        _, dvec, rvec = lax.fori_loop(
            0, _K, round_body, (tuple(dch), zeros, zeros))

        sd = _vsqrt(dvec)
        fscr[...] = sd
        dmax = plsc.load_gather(fscr, [jnp.broadcast_to(jnp.int32(_K - 1),
                                                        (_L,))])
        norm = sd / (dmax + jnp.float32(1e-8))
        penv = jnp.abs(jnp.abs(rvec - rii) - norm)
        lane_ok = ((iota >= 1) & (iota < _K)).astype(jnp.float32)
        return pen_acc + penv * lane_ok

    pen_vec = lax.fori_loop(0, 4, knn_point, jnp.zeros((_L,), jnp.float32))
    pen_vec = bf_sum(pen_vec)
    pen_gate = jnp.where(w < (_M + 3) // 4, jnp.float32(1.0), jnp.float32(0.0))

    def mse_body(j, accs):
        b = j * (_L * _UNROLL)
        out = []
        for u in range(_UNROLL):
            d = (yp_buf[pl.ds(b + u * _L, _L)]
                 - yt_buf[pl.ds(b + u * _L, _L)])
            out.append(accs[u] + d * d)
        return tuple(out)

    zeros = jnp.zeros((_L,), jnp.float32)
    cp0.wait()
    cp1.wait()
    accs = lax.fori_loop(0, _NITER, mse_body, (zeros,) * _UNROLL)
    acc = accs[0]
    for u in range(1, _UNROLL):
        acc = acc + accs[u]
    for k in range(_TAIL // _L):
        dt = tyb[pl.ds(k * _L, _L)] - ttb[pl.ds(k * _L, _L)]
        acc = acc + dt * dt

    lane0 = (iota == 0).astype(jnp.float32)
    contrib = acc * jnp.float32(0.8 / _N)
    contrib = contrib + pen_vec * lane0 * (
        pen_gate * jnp.float32(0.2 / (_M * (_K - 1))))
    orow[...] = contrib
    pltpu.sync_copy(orow, out_hbm.at[w])


_sc_kernel = functools.partial(
    pl.kernel,
    mesh=plsc.VectorSubcoreMesh(core_axis_name="c", subcore_axis_name="s"),
    compiler_params=pltpu.CompilerParams(needs_layout_passes=False),
    out_type=jax.ShapeDtypeStruct((_NW, _L), jnp.float32),
    scratch_types=[
        pltpu.VMEM((_CHUNK,), jnp.float32),
        pltpu.VMEM((_CHUNK,), jnp.float32),
        pltpu.VMEM((_TAIL,), jnp.float32),
        pltpu.VMEM((_TAIL,), jnp.float32),
        pltpu.VMEM((128,), jnp.int32),
        pltpu.VMEM((128,), jnp.float32),
        pltpu.VMEM((128,), jnp.float32),
        pltpu.VMEM((128,), jnp.float32),
        pltpu.VMEM((128,), jnp.float32),
        pltpu.VMEM((128,), jnp.float32),
        pltpu.VMEM((_L,), jnp.float32),
        pltpu.VMEM((_L,), jnp.int32),
        pltpu.VMEM((_L,), jnp.float32),
        pltpu.SemaphoreType.DMA,
        pltpu.SemaphoreType.DMA,
        pltpu.SemaphoreType.DMA,
    ],
)(_sc_body)


def kernel(y_pred, y_true, coordinates):
    idxc = jnp.asarray(_IDX_PAD)
    cxy = coordinates[idxc].T
    partials = _sc_kernel(y_pred, y_true, cxy, idxc)
    return jnp.sum(partials)


# speedup vs baseline: 1.0800x; 1.0179x over previous
import functools
import numpy as np
import jax
import jax.numpy as jnp
from jax import lax
from jax.experimental import pallas as pl
from jax.experimental.pallas import tpu as pltpu
from jax.experimental.pallas import tpu_sc as plsc

_N = 1000000
_M = 100
_K = 11
_L = 16
_NCH = 7
_NW = 32
_CHUNK = 31248
_TAIL = _N - _NW * _CHUNK
_TAIL_BASE = _NW * _CHUNK
_UNROLL = 9
_NITER = _CHUNK // (_L * _UNROLL)
_NITER1 = 108
_SPLIT = _NITER1 * _L * _UNROLL
_NITER2 = _NITER - _NITER1
_REST = _CHUNK - _SPLIT
_INF = float(np.inf)

_PERM100 = np.asarray([
    168450, 920172, 395105, 263872, 221109, 78064, 858077, 520158, 330145,
    555220, 766525, 15925, 447126, 550083, 583420, 413555, 662564, 151649,
    528156, 786375, 391712, 285245, 201641, 174004, 8595, 602412, 882598,
    554843, 78597, 752653, 318832, 130130, 118778, 291031, 851275, 141787,
    69026, 509543, 678130, 987805, 152648, 243323, 177380, 689120, 816119,
    177489, 745975, 143737, 943136, 266152, 786620, 853560, 969641, 861815,
    244708, 978776, 248512, 45655, 23208, 370197, 107389, 588445, 468632,
    950894, 196375, 417740, 909312, 483740, 709397, 199205, 358812, 550817,
    671241, 541311, 454740, 76131, 105319, 123046, 186913, 746742, 891006,
    952967, 678045, 6549, 906382, 491437, 728325, 614679, 750116, 162766,
    880843, 815723, 800078, 455911, 716915, 877054, 313050, 164116, 158191,
    157157,
], dtype=np.int32)

_IDX_PAD = np.zeros((128,), dtype=np.int32)
_IDX_PAD[:_M] = _PERM100


def _vsqrt(x):
    bits = lax.bitcast_convert_type(x, jnp.int32)
    s = lax.bitcast_convert_type(
        (bits >> 1) + jnp.int32(0x1FBD1DF5), jnp.float32)
    for _ in range(3):
        s = jnp.float32(0.5) * (s + x / s)
    return s


def _lex_take(k, i, k2, i2):
    take = (k2 < k) | ((k2 == k) & (i2 < i))
    return jnp.where(take, k2, k), jnp.where(take, i2, i)


def _sc_body(yp_hbm, yt_hbm, cxy_hbm, idx_hbm, out_hbm,
             yp_buf, yt_buf, tyb, ttb, idx_v,
             yps, yts, cxs, cys, rsub, fscr, iscr, orow, sem0, sem1, sem2):
    c = lax.axis_index("c")
    s = lax.axis_index("s")
    w = s * 2 + c
    base = w * _CHUNK

    pltpu.sync_copy(idx_hbm, idx_v)
    g0 = pltpu.async_copy(yp_hbm.at[idx_v], yps, sem2)
    g1 = pltpu.async_copy(yt_hbm.at[idx_v], yts, sem2)
    pltpu.sync_copy(cxy_hbm.at[0], cxs)
    pltpu.sync_copy(cxy_hbm.at[1], cys)

    cp0 = pltpu.async_copy(yp_hbm.at[pl.ds(base, _CHUNK)], yp_buf, sem0)
    cp1 = pltpu.async_copy(yt_hbm.at[pl.ds(base, _CHUNK)], yt_buf, sem1)

    zero16 = jnp.zeros((_L,), jnp.float32)
    for k in range(_TAIL // _L):
        tyb[pl.ds(k * _L, _L)] = zero16
        ttb[pl.ds(k * _L, _L)] = zero16

    @pl.when(w == _NW - 1)
    def _():
        pltpu.sync_copy(yp_hbm.at[pl.ds(_TAIL_BASE, _TAIL)], tyb)
        pltpu.sync_copy(yt_hbm.at[pl.ds(_TAIL_BASE, _TAIL)], ttb)

    g0.wait()
    g1.wait()

    iota = lax.iota(jnp.int32, _L)
    cxv, cyv, gidx = [], [], []
    for ci in range(_NCH):
        sl = pl.ds(ci * _L, _L)
        rsub[sl] = yps[sl] - yts[sl]
        gidx.append(iota + ci * _L)
        cxv.append(cxs[sl])
        cyv.append(cys[sl])
    rsub[pl.ds(_NCH * _L, _L)] = zero16

    def bf_sum(v):
        for sh in (8, 4, 2, 1):
            fscr[...] = v
            v = v + plsc.load_gather(fscr, [iota ^ sh])
        return v

    def bf_lex_min(k, i):
        for sh in (8, 4, 2, 1):
            fscr[...] = k
            iscr[...] = i
            idx = iota ^ sh
            k2 = plsc.load_gather(fscr, [idx])
            i2 = plsc.load_gather(iscr, [idx])
            k, i = _lex_take(k, i, k2, i2)
        return k, i

    def splat(ref, j):
        return plsc.load_gather(ref, [jnp.broadcast_to(j, (_L,))])

    def knn_point(t, pen_acc):
        ii = w * 4 + t
        cxi = splat(cxs, ii)
        cyi = splat(cys, ii)
        rii = splat(rsub, ii)

        dch = []
        for ci in range(_NCH):
            dx = cxv[ci] - cxi
            dy = cyv[ci] - cyi
            d2 = dx * dx + dy * dy
            dch.append(jnp.where(gidx[ci] < _M, d2, _INF))

        def round_body(rnd, st):
            dchs, dvec, rvec = st
            mk, mi = dchs[0], gidx[0]
            for ci in range(1, _NCH):
                mk, mi = _lex_take(mk, mi, dchs[ci], gidx[ci])
            m = jnp.min(mk)
            jstar = jnp.min(jnp.where(mk == m, mi, jnp.int32(1 << 30)))
            rstar = splat(rsub, jnp.broadcast_to(jstar, (_L,)))
            lane_hit = iota == rnd
            dvec = jnp.where(lane_hit, m, dvec)
            rvec = jnp.where(lane_hit, rstar, rvec)
            new_dchs = tuple(
                jnp.where(gidx[ci] == jstar, _INF, dchs[ci])
                for ci in range(_NCH))
            return (new_dchs, dvec, rvec)

        zeros = jnp.zeros((_L,), jnp.float32)
        _, dvec, rvec = lax.fori_loop(
            0, _K, round_body, (tuple(dch), zeros, zeros))

        sd = _vsqrt(dvec)
        fscr[...] = sd
        dmax = plsc.load_gather(fscr, [jnp.broadcast_to(jnp.int32(_K - 1),
                                                        (_L,))])
        norm = sd / (dmax + jnp.float32(1e-8))
        penv = jnp.abs(jnp.abs(rvec - rii) - norm)
        lane_ok = ((iota >= 1) & (iota < _K)).astype(jnp.float32)
        return pen_acc + penv * lane_ok

    pen_vec = lax.fori_loop(0, 4, knn_point, jnp.zeros((_L,), jnp.float32))
    pen_vec = jnp.broadcast_to(jnp.sum(pen_vec), (_L,))
    pen_gate = jnp.where(w < (_M + 3) // 4, jnp.float32(1.0), jnp.float32(0.0))

    def mse_body(j, accs):
        b = j * (_L * _UNROLL)
        out = []
        for u in range(_UNROLL):
            d = (yp_buf[pl.ds(b + u * _L, _L)]
                 - yt_buf[pl.ds(b + u * _L, _L)])
            out.append(accs[u] + d * d)
        return tuple(out)

    zeros = jnp.zeros((_L,), jnp.float32)
    cp0.wait()
    cp1.wait()
    accs = lax.fori_loop(0, _NITER, mse_body, (zeros,) * _UNROLL)
    acc = accs[0]
    for u in range(1, _UNROLL):
        acc = acc + accs[u]
    for k in range(_TAIL // _L):
        dt = tyb[pl.ds(k * _L, _L)] - ttb[pl.ds(k * _L, _L)]
        acc = acc + dt * dt

    lane0 = (iota == 0).astype(jnp.float32)
    contrib = acc * jnp.float32(0.8 / _N)
    contrib = contrib + pen_vec * lane0 * (
        pen_gate * jnp.float32(0.2 / (_M * (_K - 1))))
    orow[...] = contrib
    pltpu.sync_copy(orow, out_hbm.at[w])


_sc_kernel = functools.partial(
    pl.kernel,
    mesh=plsc.VectorSubcoreMesh(core_axis_name="c", subcore_axis_name="s"),
    compiler_params=pltpu.CompilerParams(needs_layout_passes=False),
    out_type=jax.ShapeDtypeStruct((_NW, _L), jnp.float32),
    scratch_types=[
        pltpu.VMEM((_CHUNK,), jnp.float32),
        pltpu.VMEM((_CHUNK,), jnp.float32),
        pltpu.VMEM((_TAIL,), jnp.float32),
        pltpu.VMEM((_TAIL,), jnp.float32),
        pltpu.VMEM((128,), jnp.int32),
        pltpu.VMEM((128,), jnp.float32),
        pltpu.VMEM((128,), jnp.float32),
        pltpu.VMEM((128,), jnp.float32),
        pltpu.VMEM((128,), jnp.float32),
        pltpu.VMEM((128,), jnp.float32),
        pltpu.VMEM((_L,), jnp.float32),
        pltpu.VMEM((_L,), jnp.int32),
        pltpu.VMEM((_L,), jnp.float32),
        pltpu.SemaphoreType.DMA,
        pltpu.SemaphoreType.DMA,
        pltpu.SemaphoreType.DMA,
    ],
)(_sc_body)


def kernel(y_pred, y_true, coordinates):
    idxc = jnp.asarray(_IDX_PAD)
    cxy = coordinates[idxc].T
    partials = _sc_kernel(y_pred, y_true, cxy, idxc)
    return jnp.sum(partials)
